# baseline (speedup 1.0000x reference)
import functools

import jax
import jax.numpy as jnp
from jax import lax
from jax.experimental import pallas as pl
from jax.experimental.pallas import tpu as pltpu
from jax.experimental.pallas import tpu_sc as plsc

_BATCH = 16384
_SEQ = 50
_D = 64
_V = 1000003
_VP = 1000008
_DP = 128
_NC = 2
_NS = 16
_NW = _NC * _NS
_BB = 128
_SBLK = _BATCH // _BB
_BLOCKS = _SEQ * _SBLK
_BPW = _BLOCKS // _NW
_NBUF = 2


@functools.partial(
    pl.kernel,
    out_type=jax.ShapeDtypeStruct((_SEQ, _D, _BATCH), jnp.float32),
    mesh=plsc.VectorSubcoreMesh(core_axis_name="c", subcore_axis_name="s"),
    compiler_params=pltpu.CompilerParams(needs_layout_passes=False),
    scratch_types=[
        pltpu.VMEM((_BB,), jnp.int32),
        pltpu.VMEM((_BB,), jnp.int32),
        pltpu.VMEM((_BB, _DP), jnp.float32),
        pltpu.VMEM((_BB, _DP), jnp.float32),
        pltpu.VMEM((_D, _BB), jnp.float32),
        pltpu.SemaphoreType.DMA,
        pltpu.SemaphoreType.DMA,
    ],
)
def _embed_gather(idx_hbm, table_hbm, out_hbm,
                  idx0, idx1, rows0, rows1, tile_v, sem0, sem1):
    idx_v = [idx0, idx1]
    rows_v = [rows0, rows1]
    sems = [sem0, sem1]

    wid = lax.axis_index("s") * _NC + lax.axis_index("c")
    base = wid * _BPW

    lane = lax.iota(jnp.int32, 16)

    def _start(b, g):
        s = g // _SBLK
        b0 = (g % _SBLK) * _BB
        pltpu.sync_copy(idx_hbm.at[s, pl.ds(b0, _BB)], idx_v[b])
        pltpu.make_async_copy(
            table_hbm.at[idx_v[b]], rows_v[b], sems[b]).start()

    def _finish(b, g):
        s = g // _SBLK
        b0 = (g % _SBLK) * _BB
        pltpu.make_async_copy(
            table_hbm.at[idx_v[b]], rows_v[b], sems[b]).wait()

        rv = rows_v[b]

        def _trans_row(d, carry):
            dvec = jnp.full((16,), d, dtype=jnp.int32)
            for j0 in range(0, _BB, 16):
                vals = plsc.load_gather(rv, [lane + j0, dvec])
                tile_v[d, pl.ds(j0, 16)] = vals
            return carry

        lax.fori_loop(0, _D, _trans_row, 0)
        pltpu.sync_copy(tile_v, out_hbm.at[s, :, pl.ds(b0, _BB)])

    for b in range(_NBUF):
        _start(b, base + b)

    def _step(t, carry):
        g = base + t

        @pl.when(lax.rem(t, 2) == 0)
        def _():
            _finish(0, g)
            _start(0, g + _NBUF)

        @pl.when(lax.rem(t, 2) == 1)
        def _():
            _finish(1, g)
            _start(1, g + _NBUF)
        return carry

    lax.fori_loop(0, _BPW - _NBUF, _step, 0)
    for b in range(_NBUF):
        _finish(b, base + _BPW - _NBUF + b)


def kernel(sentence, table):
    idx_t = jnp.swapaxes(sentence, 0, 1).astype(jnp.int32)
    table_p = jnp.pad(table, ((0, _VP - _V), (0, _DP - _D)))
    out = _embed_gather(idx_t, table_p)
    return jnp.transpose(out, (2, 0, 1))
